# Initial kernel scaffold; baseline (speedup 1.0000x reference)
import jax
import jax.numpy as jnp
from jax.experimental import pallas as pl


def kernel(X, X_train, y_train, W_lin, b_lin, W_e1, b_e1, W_e2, b_e2, g_mix, bn_mix, W_K, b_K, E_label, W_t1, b_t1, W_t2, g_p, bn_p, W_p1, b_p1, W_p2, b_p2, g_h, bn_h, W_head, b_head):
    raise NotImplementedError("write your pallas kernel here")


# speedup vs baseline: 12.9436x; 12.9436x over previous
import functools

import jax
import jax.numpy as jnp
from jax import lax
from jax.experimental import pallas as pl
from jax.experimental.pallas import tpu as pltpu
from jax.experimental.pallas import tpu_sc as plsc

B = 1024
F_NUM = 128
N_TRAIN = 50000
D = 256
DB = 512
C = 96

DA = 272
TILE = 512
NSTEP = 98
NP = TILE * NSTEP
SCH = 7168
NCH = NP // SCH
CAP = 4096
BIG = 3.0e38

NC = 2
NS = 16
NW = NC * NS
QPW = B // NW


def _encode_math(x, W_lin, b_lin, W_e1, b_e1, W_e2, b_e2, g, bn, W_K, b_K):
    xx = jnp.dot(x, W_lin, preferred_element_type=jnp.float32) + b_lin
    h = jnp.maximum(jnp.dot(xx, W_e1, preferred_element_type=jnp.float32) + b_e1, 0.0)
    h = jnp.dot(h, W_e2, preferred_element_type=jnp.float32) + b_e2
    xx = xx + h
    mu = jnp.mean(xx, axis=-1, keepdims=True)
    var = jnp.mean((xx - mu) * (xx - mu), axis=-1, keepdims=True)
    ln = (xx - mu) / jnp.sqrt(var + 1e-5) * g + bn
    k = jnp.dot(ln, W_K, preferred_element_type=jnp.float32) + b_K
    return xx, k


def _enc_train_body(x_ref, wl, bl, we1, be1, we2, be2, g, bn, wk, bk,
                    cka_ref, ckg_ref):
    _, k = _encode_math(x_ref[...], wl[...], bl[...], we1[...], be1[...],
                        we2[...], be2[...], g[...], bn[...], wk[...], bk[...])
    cn = jnp.sum(k * k, axis=1, keepdims=True)
    cka_ref[...] = jnp.concatenate(
        [k, cn, jnp.zeros((TILE, DA - D - 1), jnp.float32)], axis=1)
    ckg_ref[...] = k


def _enc_q_body(x_ref, wl, bl, we1, be1, we2, be2, g, bn, wk, bk,
                x_out, k_out):
    xx, k = _encode_math(x_ref[...], wl[...], bl[...], we1[...], be1[...],
                         we2[...], be2[...], g[...], bn[...], wk[...], bk[...])
    x_out[...] = xx
    k_out[...] = k


def _scores_body(k_ref, ck_ref, s_ref, tau_ref, acc_ref, NB):
    i = pl.program_id(0)
    k = k_ref[...]
    kaug = jnp.concatenate(
        [-2.0 * k, jnp.ones((NB, 1), jnp.float32),
         jnp.zeros((NB, DA - D - 1), jnp.float32)], axis=1)
    s = jnp.dot(kaug, ck_ref[...].T,
                preferred_element_type=jnp.float32)
    col = i * TILE + lax.broadcasted_iota(jnp.int32, (NB, TILE), 1)
    s = jnp.where(col < N_TRAIN, s, BIG)
    s_ref[...] = s
    tmin = jnp.min(s, axis=1, keepdims=True)

    @pl.when(i == 0)
    def _():
        acc_ref[...] = tmin

    @pl.when(i > 0)
    def _():
        acc_ref[...] = jnp.maximum(acc_ref[...], tmin)

    tau_ref[...] = jnp.broadcast_to(acc_ref[...], (NB, 16))


def _tail_body(x_ref, k_ref, ctx_ref, y_ref, e_ref, wt1, bt1, wt2,
               gp, bnp, wp1, bp1, wp2, bp2, gh, bnh, wh, bh, out_ref, QB):
    x = x_ref[...]
    k = k_ref[...]
    ctx = ctx_ref[...]
    yv = y_ref[:, :C]
    diff3 = k[:, None, :] - ctx
    sims = -jnp.sum(diff3 * diff3, axis=2)
    m = jnp.max(sims, axis=1, keepdims=True)
    e = jnp.exp(sims - m)
    p = e / jnp.sum(e, axis=1, keepdims=True)

    diff2 = diff3.reshape(QB * C, D)
    h1 = jnp.maximum(jnp.dot(diff2, wt1[...], preferred_element_type=jnp.float32)
                     + bt1[...], 0.0)
    t = jnp.dot(h1, wt2[...], preferred_element_type=jnp.float32)
    wY = jnp.sum(p * yv, axis=1, keepdims=True)
    E = e_ref[...]
    mixA = E[0][None, :] + wY * (E[1] - E[0])[None, :]
    pt = p.reshape(QB * C, 1) * t
    mixT = jnp.sum(pt.reshape(QB, C, D), axis=1)
    x = x + mixA + mixT

    mu = jnp.mean(x, axis=-1, keepdims=True)
    var = jnp.mean((x - mu) * (x - mu), axis=-1, keepdims=True)
    ln = (x - mu) / jnp.sqrt(var + 1e-5) * gp[...] + bnp[...]
    h = jnp.maximum(jnp.dot(ln, wp1[...], preferred_element_type=jnp.float32)
                    + bp1[...], 0.0)
    h = jnp.dot(h, wp2[...], preferred_element_type=jnp.float32) + bp2[...]
    x = x + h

    mu = jnp.mean(x, axis=-1, keepdims=True)
    var = jnp.mean((x - mu) * (x - mu), axis=-1, keepdims=True)
    ln = (x - mu) / jnp.sqrt(var + 1e-5) * gh[...] + bnh[...]
    out_ref[...] = (jnp.dot(jnp.maximum(ln, 0.0), wh[...],
                            preferred_element_type=jnp.float32) + bh[...])


def _f1(i):
    return (0,)


def _f2(i):
    return (0, 0)


def _mono_u32(v):
    b = plsc.bitcast(v, jnp.int32)
    m = b ^ ((b >> 31) | jnp.int32(-2147483648))
    return plsc.bitcast(m, jnp.uint32)


def _sc_body(scores_hbm, tau_hbm, ck_hbm, y_hbm, ctx_hbm, yout_hbm,
             tau_v, sbuf_a, sbuf_b, val_v, idx_v, sel_v, rows_v, y_v, ybuf,
             sem, sem_a, sem_b, qpw):
    wid = lax.axis_index("s") * NC + lax.axis_index("c")
    qbase = wid * qpw
    lanes = lax.iota(jnp.int32, 16)
    z16 = jnp.zeros((16,), jnp.int32)
    pltpu.sync_copy(y_hbm, y_v)

    def per_q(qi, _):
        q = qbase + qi
        pltpu.sync_copy(tau_hbm.at[q], tau_v)
        tau16 = tau_v[...]

        bufs = (sbuf_a, sbuf_b)
        sems = (sem_a, sem_b)
        handles = [None] * NCH
        handles[0] = pltpu.async_copy(scores_hbm.at[q * NCH], bufs[0], sems[0])
        offv = z16
        for ch in range(NCH):
            if ch + 1 < NCH:
                handles[ch + 1] = pltpu.async_copy(
                    scores_hbm.at[q * NCH + (ch + 1)], bufs[(ch + 1) % 2],
                    sems[(ch + 1) % 2])
            handles[ch].wait()
            buf = bufs[ch % 2]

            @plsc.parallel_loop(0, SCH // 16, unroll=4, carry=offv)
            def vloop(i, off):
                v = buf[pl.ds(i * 16, 16)]
                msk = v <= tau16
                ones = jnp.where(msk, 1, 0).astype(jnp.int32)
                pref = plsc.cumsum(ones)
                pos = off + pref - 1
                gidx = jnp.full((16,), ch * SCH + i * 16, jnp.int32) + lanes
                keep = jnp.logical_and(msk, pos < CAP)
                plsc.store_scatter(val_v, [pos], v, mask=keep)
                plsc.store_scatter(idx_v, [pos], gidx, mask=keep)
                return off + plsc.all_reduce_population_count(msk)

            offv = vloop
        M = jnp.max(offv)
        Mc = jnp.minimum(M, CAP - 16)
        plsc.store_scatter(val_v, [jnp.full((16,), Mc, jnp.int32) + lanes],
                           jnp.full((16,), BIG, jnp.float32))
        nv = (M + 15) // 16

        def count_lt(t16):
            @plsc.parallel_loop(0, nv, unroll=4, carry=z16)
            def cloop(i, acc):
                mu = _mono_u32(val_v[pl.ds(i * 16, 16)])
                return acc + plsc.all_reduce_population_count(mu < t16)
            return cloop

        def bit_body(j, kap):
            bitv = jnp.full((16,), 1, jnp.uint32) << (
                jnp.full((16,), 31, jnp.uint32) - j.astype(jnp.uint32))
            trial = kap | bitv
            cnt = count_lt(trial)
            return jnp.where(cnt <= C - 1, trial, kap)

        kap = lax.fori_loop(0, 32, bit_body, jnp.zeros((16,), jnp.uint32))
        n1 = count_lt(kap)

        def selb(i, carry):
            offlt, offeq = carry
            mu = _mono_u32(val_v[pl.ds(i * 16, 16)])
            iv = idx_v[pl.ds(i * 16, 16)]
            ltm = mu < kap
            eqm = mu == kap
            plt = plsc.cumsum(jnp.where(ltm, 1, 0).astype(jnp.int32))
            peq = plsc.cumsum(jnp.where(eqm, 1, 0).astype(jnp.int32))
            pos = jnp.where(ltm, offlt + plt - 1, n1 + offeq + peq - 1)
            keep = jnp.logical_and(jnp.logical_or(ltm, eqm), pos < C)
            plsc.store_scatter(sel_v, [pos], iv, mask=keep)
            return (offlt + plsc.all_reduce_population_count(ltm),
                    offeq + plsc.all_reduce_population_count(eqm))

        lax.fori_loop(0, nv, selb, (z16, z16))

        gather = pltpu.async_copy(ck_hbm.at[sel_v.at[pl.ds(0, C)]],
                                  rows_v, sem)
        for i in range(8):
            if i < C // 16:
                iv = sel_v[pl.ds(i * 16, 16)]
                yg = plsc.load_gather(y_v, [iv]).astype(jnp.float32)
                ybuf[pl.ds(i * 16, 16)] = yg
            else:
                ybuf[pl.ds(i * 16, 16)] = jnp.zeros((16,), jnp.float32)
        pltpu.sync_copy(ybuf, yout_hbm.at[q])
        gather.wait()
        pltpu.sync_copy(rows_v, ctx_hbm.at[q])
        return 0

    lax.fori_loop(0, qpw, per_q, 0)


def _sc_select(scores2, tau, ckg, ypad):
    nb = tau.shape[0]
    mesh = plsc.VectorSubcoreMesh(core_axis_name="c", subcore_axis_name="s")
    return pl.kernel(
        functools.partial(_sc_body, qpw=nb // NW),
        out_type=[jax.ShapeDtypeStruct((nb, C, D), jnp.float32),
                  jax.ShapeDtypeStruct((nb, 128), jnp.float32)],
        mesh=mesh,
        scratch_types=[pltpu.VMEM((16,), jnp.float32),
                       pltpu.VMEM((SCH,), jnp.float32),
                       pltpu.VMEM((SCH,), jnp.float32),
                       pltpu.VMEM((CAP,), jnp.float32),
                       pltpu.VMEM((CAP,), jnp.int32),
                       pltpu.VMEM((128,), jnp.int32),
                       pltpu.VMEM((C, D), jnp.float32),
                       pltpu.VMEM((NP,), jnp.int32),
                       pltpu.VMEM((128,), jnp.float32),
                       pltpu.SemaphoreType.DMA,
                       pltpu.SemaphoreType.DMA,
                       pltpu.SemaphoreType.DMA],
        compiler_params=pltpu.CompilerParams(needs_layout_passes=False),
    )(scores2, tau, ckg, ypad)


def kernel(X, X_train, y_train, W_lin, b_lin, W_e1, b_e1, W_e2, b_e2,
           g_mix, bn_mix, W_K, b_K, E_label, W_t1, b_t1, W_t2,
           g_p, bn_p, W_p1, b_p1, W_p2, b_p2, g_h, bn_h, W_head, b_head):
    Xp = jnp.pad(X_train, ((0, NP - N_TRAIN), (0, 0)))
    yp = jnp.pad(y_train.astype(jnp.int32), (0, NP - N_TRAIN))
    del X_train, y_train

    wspecs = [pl.BlockSpec((F_NUM, D), _f2), pl.BlockSpec((D,), _f1),
              pl.BlockSpec((D, DB), _f2), pl.BlockSpec((DB,), _f1),
              pl.BlockSpec((DB, D), _f2), pl.BlockSpec((D,), _f1),
              pl.BlockSpec((D,), _f1), pl.BlockSpec((D,), _f1),
              pl.BlockSpec((D, D), _f2), pl.BlockSpec((D,), _f1)]
    enc_w = (W_lin, b_lin, W_e1, b_e1, W_e2, b_e2, g_mix, bn_mix, W_K, b_K)

    cka, ckg = pl.pallas_call(
        _enc_train_body,
        grid=(NSTEP,),
        in_specs=[pl.BlockSpec((TILE, F_NUM), lambda i: (i, 0))] + wspecs,
        out_specs=[pl.BlockSpec((TILE, DA), lambda i: (i, 0)),
                   pl.BlockSpec((TILE, D), lambda i: (i, 0))],
        out_shape=[jax.ShapeDtypeStruct((NP, DA), jnp.float32),
                   jax.ShapeDtypeStruct((NP, D), jnp.float32)],
    )(Xp, *enc_w)

    xq, kq = pl.pallas_call(
        _enc_q_body,
        grid=(B // TILE,),
        in_specs=[pl.BlockSpec((TILE, F_NUM), lambda i: (i, 0))] + wspecs,
        out_specs=[pl.BlockSpec((TILE, D), lambda i: (i, 0)),
                   pl.BlockSpec((TILE, D), lambda i: (i, 0))],
        out_shape=[jax.ShapeDtypeStruct((B, D), jnp.float32),
                   jax.ShapeDtypeStruct((B, D), jnp.float32)],
    )(X, *enc_w)

    HB = B // 2
    QB = 64
    tail_w = [pl.BlockSpec((2, D), _f2),
              pl.BlockSpec((D, DB), _f2), pl.BlockSpec((DB,), _f1),
              pl.BlockSpec((DB, D), _f2),
              pl.BlockSpec((D,), _f1), pl.BlockSpec((D,), _f1),
              pl.BlockSpec((D, DB), _f2), pl.BlockSpec((DB,), _f1),
              pl.BlockSpec((DB, D), _f2), pl.BlockSpec((D,), _f1),
              pl.BlockSpec((D,), _f1), pl.BlockSpec((D,), _f1),
              pl.BlockSpec((D, 2), _f2), pl.BlockSpec((2,), _f1)]
    outs = []
    for h in range(2):
        kh = lax.slice_in_dim(kq, h * HB, (h + 1) * HB, axis=0)
        xh = lax.slice_in_dim(xq, h * HB, (h + 1) * HB, axis=0)
        scores, tau = pl.pallas_call(
            functools.partial(_scores_body, NB=HB),
            grid=(NSTEP,),
            in_specs=[pl.BlockSpec((HB, D), lambda i: (0, 0)),
                      pl.BlockSpec((TILE, DA), lambda i: (i, 0))],
            out_specs=[pl.BlockSpec((HB, TILE), lambda i: (0, i)),
                       pl.BlockSpec((HB, 16), lambda i: (0, 0))],
            out_shape=[jax.ShapeDtypeStruct((HB, NP), jnp.float32),
                       jax.ShapeDtypeStruct((HB, 16), jnp.float32)],
            scratch_shapes=[pltpu.VMEM((HB, 1), jnp.float32)],
        )(kh, cka)

        ctx, yctx = _sc_select(scores.reshape(HB * NCH, SCH), tau, ckg, yp)

        outs.append(pl.pallas_call(
            functools.partial(_tail_body, QB=QB),
            grid=(HB // QB,),
            in_specs=[pl.BlockSpec((QB, D), lambda i: (i, 0)),
                      pl.BlockSpec((QB, D), lambda i: (i, 0)),
                      pl.BlockSpec((QB, C, D), lambda i: (i, 0, 0)),
                      pl.BlockSpec((QB, 128), lambda i: (i, 0))] + tail_w,
            out_specs=pl.BlockSpec((QB, 2), lambda i: (i, 0)),
            out_shape=jax.ShapeDtypeStruct((HB, 2), jnp.float32),
        )(xh, kh, ctx, yctx, E_label, W_t1, b_t1, W_t2, g_p, bn_p,
          W_p1, b_p1, W_p2, b_p2, g_h, bn_h, W_head, b_head))
    return jnp.concatenate(outs, axis=0)
